# baseline (speedup 1.0000x reference)
import jax
import jax.numpy as jnp
from jax.experimental import pallas as pl
from jax.experimental.pallas import tpu as pltpu

M, B, D = 65536, 8192, 256
T = M + B
CH = 8192
NBUF = 6
N_OLD = M // CH
N_NEW = B // CH


def _dma_body(k, v, nk, nv, ok, ov, *rest):
    bufs = rest[:NBUF]
    in_sems, out_sems = rest[NBUF], rest[NBUF + 1]

    chunks = []
    for c in range(N_OLD):
        chunks.append((k, c * CH, ok, c * CH))
    for c in range(N_NEW):
        chunks.append((nk, c * CH, ok, M + c * CH))
    for c in range(N_OLD):
        chunks.append((v, c * CH, ov, c * CH))
    for c in range(N_NEW):
        chunks.append((nv, c * CH, ov, M + c * CH))
    n = len(chunks)

    in_cp = [None] * n
    out_cp = [None] * n

    def start_in(i):
        src, soff, _, _ = chunks[i]
        b = i % NBUF
        in_cp[i] = pltpu.make_async_copy(
            src.at[pl.ds(soff, CH), :], bufs[b], in_sems.at[b])
        in_cp[i].start()

    def start_out(i):
        _, _, dst, doff = chunks[i]
        b = i % NBUF
        out_cp[i] = pltpu.make_async_copy(
            bufs[b], dst.at[pl.ds(doff, CH), :], out_sems.at[b])
        out_cp[i].start()

    for i in range(min(NBUF, n)):
        start_in(i)
    for i in range(n):
        in_cp[i].wait()
        start_out(i)
        j = i + NBUF - 1
        if i >= 1 and j < n:
            out_cp[i - 1].wait()
            start_in(j)
    for i in range(max(0, n - NBUF), n):
        out_cp[i].wait()


def kernel(keys, vals, new_keys, new_vals):
    hbm = pl.BlockSpec(memory_space=pltpu.MemorySpace.HBM)
    out_shape = jax.ShapeDtypeStruct((T, D), keys.dtype)
    scratch = [pltpu.VMEM((CH, D), keys.dtype) for _ in range(NBUF)]
    scratch += [pltpu.SemaphoreType.DMA((NBUF,)),
                pltpu.SemaphoreType.DMA((NBUF,))]
    return pl.pallas_call(
        _dma_body,
        in_specs=[hbm, hbm, hbm, hbm],
        out_specs=[hbm, hbm],
        out_shape=[out_shape, out_shape],
        scratch_shapes=scratch,
    )(keys, vals, new_keys, new_vals)


# speedup vs baseline: 1.0028x; 1.0028x over previous
import jax
import jax.numpy as jnp
from jax.experimental import pallas as pl
from jax.experimental.pallas import tpu as pltpu

M, B, D = 65536, 8192, 256
T = M + B
CH = 8192
NBUF = 7
N_OLD = M // CH
N_NEW = B // CH


def _dma_body(k, v, nk, nv, ok, ov, *rest):
    bufs = rest[:NBUF]
    in_sems, out_sems = rest[NBUF], rest[NBUF + 1]

    chunks = []
    for c in range(N_OLD):
        chunks.append((k, c * CH, ok, c * CH))
        chunks.append((v, c * CH, ov, c * CH))
    for c in range(N_NEW):
        chunks.append((nk, c * CH, ok, M + c * CH))
        chunks.append((nv, c * CH, ov, M + c * CH))
    n = len(chunks)

    in_cp = [None] * n
    out_cp = [None] * n

    def start_in(i):
        src, soff, _, _ = chunks[i]
        b = i % NBUF
        in_cp[i] = pltpu.make_async_copy(
            src.at[pl.ds(soff, CH), :], bufs[b], in_sems.at[b])
        in_cp[i].start()

    def start_out(i):
        _, _, dst, doff = chunks[i]
        b = i % NBUF
        out_cp[i] = pltpu.make_async_copy(
            bufs[b], dst.at[pl.ds(doff, CH), :], out_sems.at[b])
        out_cp[i].start()

    for i in range(min(NBUF, n)):
        start_in(i)
    for i in range(n):
        in_cp[i].wait()
        start_out(i)
        j = i + NBUF - 1
        if i >= 1 and j < n:
            out_cp[i - 1].wait()
            start_in(j)
    for i in range(max(0, n - NBUF), n):
        out_cp[i].wait()


def kernel(keys, vals, new_keys, new_vals):
    hbm = pl.BlockSpec(memory_space=pltpu.MemorySpace.HBM)
    out_shape = jax.ShapeDtypeStruct((T, D), keys.dtype)
    scratch = [pltpu.VMEM((CH, D), keys.dtype) for _ in range(NBUF)]
    scratch += [pltpu.SemaphoreType.DMA((NBUF,)),
                pltpu.SemaphoreType.DMA((NBUF,))]
    return pl.pallas_call(
        _dma_body,
        in_specs=[hbm, hbm, hbm, hbm],
        out_specs=[hbm, hbm],
        out_shape=[out_shape, out_shape],
        scratch_shapes=scratch,
    )(keys, vals, new_keys, new_vals)


# speedup vs baseline: 1.0037x; 1.0009x over previous
import jax
import jax.numpy as jnp
from jax.experimental import pallas as pl
from jax.experimental.pallas import tpu as pltpu

M, B, D = 65536, 8192, 256
T = M + B
CH = 8192
NBUF = 6
N_OLD = M // CH
N_NEW = B // CH


def _dma_body(k, v, nk, nv, ok, ov, *rest):
    bufs = rest[:NBUF]
    in_sems, out_sems = rest[NBUF], rest[NBUF + 1]

    chunks = []
    for c in range(N_OLD):
        chunks.append((k, c * CH, ok, c * CH))
        chunks.append((v, c * CH, ov, c * CH))
    for c in range(N_NEW):
        chunks.append((nk, c * CH, ok, M + c * CH))
        chunks.append((nv, c * CH, ov, M + c * CH))
    n = len(chunks)

    in_cp = [None] * n
    out_cp = [None] * n

    def start_in(i):
        src, soff, _, _ = chunks[i]
        b = i % NBUF
        in_cp[i] = pltpu.make_async_copy(
            src.at[pl.ds(soff, CH), :], bufs[b], in_sems.at[b])
        in_cp[i].start()

    def start_out(i):
        _, _, dst, doff = chunks[i]
        b = i % NBUF
        out_cp[i] = pltpu.make_async_copy(
            bufs[b], dst.at[pl.ds(doff, CH), :], out_sems.at[b])
        out_cp[i].start()

    for i in range(min(NBUF, n)):
        start_in(i)
    for i in range(n):
        in_cp[i].wait()
        start_out(i)
        j = i + NBUF - 1
        if i >= 1 and j < n:
            out_cp[i - 1].wait()
            start_in(j)
    for i in range(max(0, n - NBUF), n):
        out_cp[i].wait()


def kernel(keys, vals, new_keys, new_vals):
    hbm = pl.BlockSpec(memory_space=pltpu.MemorySpace.HBM)
    out_shape = jax.ShapeDtypeStruct((T, D), keys.dtype)
    scratch = [pltpu.VMEM((CH, D), keys.dtype) for _ in range(NBUF)]
    scratch += [pltpu.SemaphoreType.DMA((NBUF,)),
                pltpu.SemaphoreType.DMA((NBUF,))]
    return pl.pallas_call(
        _dma_body,
        in_specs=[hbm, hbm, hbm, hbm],
        out_specs=[hbm, hbm],
        out_shape=[out_shape, out_shape],
        scratch_shapes=scratch,
    )(keys, vals, new_keys, new_vals)
